# baseline (speedup 1.0000x reference)
import functools

import jax
import jax.numpy as jnp
from jax import lax
from jax.experimental import pallas as pl
from jax.experimental.pallas import tpu as pltpu
from jax.experimental.pallas import tpu_sc as plsc

N = 10000
E = 320000
D_IN = 128
H1 = 64
H2 = 32
W = 128

NC = 2
NS = 16
NW = NC * NS
CHUNK = 128
BLK = 8
NBLK = 10
NCHUNK = BLK * NBLK
E_PAD = NW * CHUNK * NCHUNK
N_PAD = 10240
RPS = N_PAD // NS
L = 16

_MESH = plsc.VectorSubcoreMesh(core_axis_name="c", subcore_axis_name="s")
_SC_PARAMS = pltpu.CompilerParams(needs_layout_passes=False)


def _deg_body(dst_hbm, out_hbm, dst_v, deg_v):
    c = lax.axis_index("c")
    s = lax.axis_index("s")
    wid = c * NS + s
    pltpu.sync_copy(dst_hbm.at[wid], dst_v)

    def zero(i, carry):
        deg_v[pl.ds(i * L, L)] = jnp.zeros((L,), jnp.float32)
        return carry

    lax.fori_loop(0, N_PAD // L, zero, 0)

    ones = jnp.ones((L,), jnp.float32)

    def body(i, carry):
        j = i // (CHUNK // L)
        k = i % (CHUNK // L)
        idx = dst_v[j, pl.ds(k * L, L)]
        plsc.addupdate_scatter(deg_v, [idx], ones)
        return carry

    lax.fori_loop(0, NCHUNK * (CHUNK // L), body, 0)
    pltpu.sync_copy(deg_v, out_hbm.at[wid])


@functools.partial(
    pl.kernel,
    mesh=_MESH,
    compiler_params=_SC_PARAMS,
    out_type=jax.ShapeDtypeStruct((NW, N_PAD), jnp.float32),
    scratch_types=[
        pltpu.VMEM((NCHUNK, CHUNK), jnp.int32),
        pltpu.VMEM((N_PAD,), jnp.float32),
    ],
)
def _deg_kernel(dst_hbm, out_hbm, dst_v, deg_v):
    _deg_body(dst_hbm, out_hbm, dst_v, deg_v)


def _spmm_body(h_hbm, src_hbm, dst_hbm, zeros_hbm, out_hbm,
               sr0, sr1, dr0, dr1, rows_a, rows_b, acc, sem_a, sem_b, sem_i):
    c = lax.axis_index("c")
    s = lax.axis_index("s")
    wid = c * NS + s

    def idx_load(blk, sr, dr):
        pltpu.async_copy(src_hbm.at[wid, pl.ds(blk * BLK, BLK)], sr, sem_i)
        pltpu.async_copy(dst_hbm.at[wid, pl.ds(blk * BLK, BLK)], dr, sem_i)

    def idx_wait(sr, dr):
        pltpu.make_async_copy(src_hbm.at[wid, pl.ds(0, BLK)], sr, sem_i).wait()
        pltpu.make_async_copy(dst_hbm.at[wid, pl.ds(0, BLK)], dr, sem_i).wait()

    idx_load(0, sr0, dr0)
    cz = pltpu.async_copy(zeros_hbm, acc.at[pl.ds(s * RPS, RPS)], sem_b)
    idx_wait(sr0, dr0)
    cz.wait()
    idx_load(1, sr1, dr1)
    plsc.subcore_barrier()

    def process(srcs, dsts):
        g = pltpu.async_copy(h_hbm.at[srcs.at[0]], rows_a, sem_a)
        for k in range(BLK):
            cur = rows_a if k % 2 == 0 else rows_b
            if k + 1 < BLK:
                nxt = rows_b if k % 2 == 0 else rows_a
                nsem = sem_b if k % 2 == 0 else sem_a
                g_next = pltpu.async_copy(h_hbm.at[srcs.at[k + 1]], nxt, nsem)
            g.wait()
            pltpu.sync_copy(cur, acc.at[dsts.at[k]], add=True)
            if k + 1 < BLK:
                g = g_next

    def body(i, carry):
        process(sr0, dr0)
        idx_wait(sr1, dr1)

        @pl.when(i < NBLK // 2 - 1)
        def _():
            idx_load(2 * i + 2, sr0, dr0)

        process(sr1, dr1)

        @pl.when(i < NBLK // 2 - 1)
        def _():
            idx_wait(sr0, dr0)
            idx_load(2 * i + 3, sr1, dr1)

        return carry

    lax.fori_loop(0, NBLK // 2, body, 0)
    plsc.subcore_barrier()
    pltpu.sync_copy(acc.at[pl.ds(s * RPS, RPS)],
                    out_hbm.at[c, pl.ds(s * RPS, RPS)])


@functools.partial(
    pl.kernel,
    mesh=_MESH,
    compiler_params=_SC_PARAMS,
    out_type=jax.ShapeDtypeStruct((NC, N_PAD, W), jnp.float32),
    scratch_types=[
        pltpu.VMEM((BLK, CHUNK), jnp.int32),
        pltpu.VMEM((BLK, CHUNK), jnp.int32),
        pltpu.VMEM((BLK, CHUNK), jnp.int32),
        pltpu.VMEM((BLK, CHUNK), jnp.int32),
        pltpu.VMEM((CHUNK, W), jnp.float32),
        pltpu.VMEM((CHUNK, W), jnp.float32),
        pltpu.VMEM_SHARED((N_PAD, W), jnp.float32),
        pltpu.SemaphoreType.DMA,
        pltpu.SemaphoreType.DMA,
        pltpu.SemaphoreType.DMA,
    ],
)
def _spmm_kernel(h_hbm, src_hbm, dst_hbm, zeros_hbm, out_hbm,
                 sr0, sr1, dr0, dr1, rows_a, rows_b, acc,
                 sem_a, sem_b, sem_i):
    _spmm_body(h_hbm, src_hbm, dst_hbm, zeros_hbm, out_hbm,
               sr0, sr1, dr0, dr1, rows_a, rows_b, acc, sem_a, sem_b, sem_i)


def _tc1_body(x_ref, w1_ref, deg_ref, h_ref, inv_ref):
    dsum = jnp.sum(deg_ref[...], axis=1, keepdims=True)
    inv = lax.rsqrt(jnp.maximum(dsum, 1.0))
    inv_ref[...] = inv
    h_ref[...] = jnp.dot(x_ref[...], w1_ref[...],
                         preferred_element_type=jnp.float32) * inv


def _tc2_body(p0_ref, p1_ref, inv_ref, w_ref, out_ref):
    inv = inv_ref[...]
    hidden = jnp.maximum((p0_ref[...] + p1_ref[...]) * inv, 0.0)
    out_ref[...] = jnp.dot(hidden, w_ref[...],
                           preferred_element_type=jnp.float32) * inv


def _tc3_body(zm0_ref, zm1_ref, zl0_ref, zl1_ref, inv_ref, eps_ref, z_ref):
    inv = inv_ref[...]
    zm = (zm0_ref[...] + zm1_ref[...]) * inv
    zl = (zl0_ref[...] + zl1_ref[...]) * inv
    z_ref[...] = zm + jnp.exp(0.5 * zl) * eps_ref[...]


def kernel(x, adj, W1, W_mu, W_sigma, epsilon):
    src = adj[0]
    dst = adj[1]
    pad = E_PAD - E
    src3 = jnp.concatenate([src, jnp.zeros((pad,), jnp.int32)]).reshape(
        NW, NCHUNK, CHUNK)
    dst3 = jnp.concatenate([dst, jnp.full((pad,), N, jnp.int32)]).reshape(
        NW, NCHUNK, CHUNK)

    zeros_h = jnp.zeros((RPS, W), jnp.float32)
    W1p = jnp.pad(W1, ((0, 0), (0, W - H1)))
    Wcat = jnp.pad(jnp.concatenate([W_mu, W_sigma], axis=1),
                   ((0, W - H1), (0, W - H1)))

    deg_parts = _deg_kernel(dst3)
    degT = deg_parts.T[:N]

    h0s, inv = pl.pallas_call(
        _tc1_body,
        out_shape=[
            jax.ShapeDtypeStruct((N, W), jnp.float32),
            jax.ShapeDtypeStruct((N, 1), jnp.float32),
        ],
    )(x, W1p, degT)

    p = _spmm_kernel(h0s, src3, dst3, zeros_h)

    h2s = pl.pallas_call(
        _tc2_body,
        out_shape=jax.ShapeDtypeStruct((N, W), jnp.float32),
    )(p[0, :N], p[1, :N], inv, Wcat)

    q = _spmm_kernel(h2s, src3, dst3, zeros_h)

    Z = pl.pallas_call(
        _tc3_body,
        out_shape=jax.ShapeDtypeStruct((N, H2), jnp.float32),
    )(q[0, :N, :H2], q[1, :N, :H2], q[0, :N, H2:H1], q[1, :N, H2:H1],
      inv, epsilon)
    return Z


# speedup vs baseline: 1.4870x; 1.4870x over previous
import functools

import jax
import jax.numpy as jnp
from jax import lax
from jax.experimental import pallas as pl
from jax.experimental.pallas import tpu as pltpu
from jax.experimental.pallas import tpu_sc as plsc

N = 10000
E = 320000
D_IN = 128
H1 = 64
H2 = 32
W = 64

NC = 2
NS = 16
NW = NC * NS
CHUNK = 128
BLK = 8
NBLK = 10
NCHUNK = BLK * NBLK
E_PAD = NW * CHUNK * NCHUNK
N_PAD = 10240
RPS = N_PAD // NS
L = 16

_MESH = plsc.VectorSubcoreMesh(core_axis_name="c", subcore_axis_name="s")
_SC_PARAMS = pltpu.CompilerParams(needs_layout_passes=False,
                                  use_tc_tiling_on_sc=False)


def _deg_body(dst_hbm, out_hbm, dst_v, deg_v):
    c = lax.axis_index("c")
    s = lax.axis_index("s")
    wid = c * NS + s
    pltpu.sync_copy(dst_hbm.at[wid], dst_v)

    def zero(i, carry):
        deg_v[pl.ds(i * L, L)] = jnp.zeros((L,), jnp.float32)
        return carry

    lax.fori_loop(0, N_PAD // L, zero, 0)

    ones = jnp.ones((L,), jnp.float32)

    def body(i, carry):
        j = i // (CHUNK // L)
        k = i % (CHUNK // L)
        idx = dst_v[j, pl.ds(k * L, L)]
        plsc.addupdate_scatter(deg_v, [idx], ones)
        return carry

    lax.fori_loop(0, NCHUNK * (CHUNK // L), body, 0)
    pltpu.sync_copy(deg_v, out_hbm.at[wid])


@functools.partial(
    pl.kernel,
    mesh=_MESH,
    compiler_params=_SC_PARAMS,
    out_type=jax.ShapeDtypeStruct((NW, N_PAD), jnp.float32),
    scratch_types=[
        pltpu.VMEM((NCHUNK, CHUNK), jnp.int32),
        pltpu.VMEM((N_PAD,), jnp.float32),
    ],
)
def _deg_kernel(dst_hbm, out_hbm, dst_v, deg_v):
    _deg_body(dst_hbm, out_hbm, dst_v, deg_v)


def _spmm_body(h_hbm, src_hbm, dst_hbm, zeros_hbm, out_hbm,
               sr0, sr1, dr0, dr1, rows_a, rows_b, acc, sem_a, sem_b, sem_i):
    c = lax.axis_index("c")
    s = lax.axis_index("s")
    wid = c * NS + s

    def idx_load(blk, sr, dr):
        pltpu.async_copy(src_hbm.at[wid, pl.ds(blk * BLK, BLK)], sr, sem_i)
        pltpu.async_copy(dst_hbm.at[wid, pl.ds(blk * BLK, BLK)], dr, sem_i)

    def idx_wait(sr, dr):
        pltpu.make_async_copy(src_hbm.at[wid, pl.ds(0, BLK)], sr, sem_i).wait()
        pltpu.make_async_copy(dst_hbm.at[wid, pl.ds(0, BLK)], dr, sem_i).wait()

    idx_load(0, sr0, dr0)
    cz = pltpu.async_copy(zeros_hbm, acc.at[pl.ds(s * RPS, RPS)], sem_b)
    idx_wait(sr0, dr0)
    cz.wait()
    idx_load(1, sr1, dr1)
    plsc.subcore_barrier()

    def process(srcs, dsts):
        g = pltpu.async_copy(h_hbm.at[srcs.at[0]], rows_a, sem_a)
        for k in range(BLK):
            cur = rows_a if k % 2 == 0 else rows_b
            if k + 1 < BLK:
                nxt = rows_b if k % 2 == 0 else rows_a
                nsem = sem_b if k % 2 == 0 else sem_a
                g_next = pltpu.async_copy(h_hbm.at[srcs.at[k + 1]], nxt, nsem)
            g.wait()
            pltpu.sync_copy(cur, acc.at[dsts.at[k]], add=True)
            if k + 1 < BLK:
                g = g_next

    def body(i, carry):
        process(sr0, dr0)
        idx_wait(sr1, dr1)

        @pl.when(i < NBLK // 2 - 1)
        def _():
            idx_load(2 * i + 2, sr0, dr0)

        process(sr1, dr1)

        @pl.when(i < NBLK // 2 - 1)
        def _():
            idx_wait(sr0, dr0)
            idx_load(2 * i + 3, sr1, dr1)

        return carry

    lax.fori_loop(0, NBLK // 2, body, 0)
    plsc.subcore_barrier()
    pltpu.sync_copy(acc.at[pl.ds(s * RPS, RPS)],
                    out_hbm.at[c, pl.ds(s * RPS, RPS)])


@functools.partial(
    pl.kernel,
    mesh=_MESH,
    compiler_params=_SC_PARAMS,
    out_type=jax.ShapeDtypeStruct((NC, N_PAD, W), jnp.float32),
    scratch_types=[
        pltpu.VMEM((BLK, CHUNK), jnp.int32),
        pltpu.VMEM((BLK, CHUNK), jnp.int32),
        pltpu.VMEM((BLK, CHUNK), jnp.int32),
        pltpu.VMEM((BLK, CHUNK), jnp.int32),
        pltpu.VMEM((CHUNK, W), jnp.float32),
        pltpu.VMEM((CHUNK, W), jnp.float32),
        pltpu.VMEM_SHARED((N_PAD, W), jnp.float32),
        pltpu.SemaphoreType.DMA,
        pltpu.SemaphoreType.DMA,
        pltpu.SemaphoreType.DMA,
    ],
)
def _spmm_kernel(h_hbm, src_hbm, dst_hbm, zeros_hbm, out_hbm,
                 sr0, sr1, dr0, dr1, rows_a, rows_b, acc,
                 sem_a, sem_b, sem_i):
    _spmm_body(h_hbm, src_hbm, dst_hbm, zeros_hbm, out_hbm,
               sr0, sr1, dr0, dr1, rows_a, rows_b, acc, sem_a, sem_b, sem_i)


def _tc1_body(x_ref, w1_ref, deg_ref, h_ref, inv_ref):
    dsum = jnp.sum(deg_ref[...], axis=1, keepdims=True)
    inv = lax.rsqrt(jnp.maximum(dsum, 1.0))
    inv_ref[...] = inv
    h_ref[...] = jnp.dot(x_ref[...], w1_ref[...],
                         preferred_element_type=jnp.float32) * inv


def _tc2_body(p0_ref, p1_ref, inv_ref, w_ref, out_ref):
    inv = inv_ref[...]
    hidden = jnp.maximum((p0_ref[...] + p1_ref[...]) * inv, 0.0)
    out_ref[...] = jnp.dot(hidden, w_ref[...],
                           preferred_element_type=jnp.float32) * inv


def _tc3_body(zm0_ref, zm1_ref, zl0_ref, zl1_ref, inv_ref, eps_ref, z_ref):
    inv = inv_ref[...]
    zm = (zm0_ref[...] + zm1_ref[...]) * inv
    zl = (zl0_ref[...] + zl1_ref[...]) * inv
    z_ref[...] = zm + jnp.exp(0.5 * zl) * eps_ref[...]


def kernel(x, adj, W1, W_mu, W_sigma, epsilon):
    src = adj[0]
    dst = adj[1]
    pad = E_PAD - E
    src3 = jnp.concatenate([src, jnp.zeros((pad,), jnp.int32)]).reshape(
        NW, NCHUNK, CHUNK)
    dst3 = jnp.concatenate([dst, jnp.full((pad,), N, jnp.int32)]).reshape(
        NW, NCHUNK, CHUNK)

    zeros_h = jnp.zeros((RPS, W), jnp.float32)
    Wcat = jnp.concatenate([W_mu, W_sigma], axis=1)

    deg_parts = _deg_kernel(dst3)
    degT = deg_parts.T[:N]

    h0s, inv = pl.pallas_call(
        _tc1_body,
        out_shape=[
            jax.ShapeDtypeStruct((N, W), jnp.float32),
            jax.ShapeDtypeStruct((N, 1), jnp.float32),
        ],
    )(x, W1, degT)

    p = _spmm_kernel(h0s, src3, dst3, zeros_h)

    h2s = pl.pallas_call(
        _tc2_body,
        out_shape=jax.ShapeDtypeStruct((N, W), jnp.float32),
    )(p[0, :N], p[1, :N], inv, Wcat)

    q = _spmm_kernel(h2s, src3, dst3, zeros_h)

    Z = pl.pallas_call(
        _tc3_body,
        out_shape=jax.ShapeDtypeStruct((N, H2), jnp.float32),
    )(q[0, :N, :H2], q[1, :N, :H2], q[0, :N, H2:H1], q[1, :N, H2:H1],
      inv, epsilon)
    return Z


# speedup vs baseline: 1.5306x; 1.0294x over previous
import functools

import jax
import jax.numpy as jnp
from jax import lax
from jax.experimental import pallas as pl
from jax.experimental.pallas import tpu as pltpu
from jax.experimental.pallas import tpu_sc as plsc

N = 10000
E = 320000
D_IN = 128
H1 = 64
H2 = 32
W = 64

NC = 2
NS = 16
NW = NC * NS
CHUNK = 128
BLK = 8
NBLK = 10
NCHUNK = BLK * NBLK
E_PAD = NW * CHUNK * NCHUNK
N_PAD = 10240
RPS = N_PAD // NS
L = 16

_MESH = plsc.VectorSubcoreMesh(core_axis_name="c", subcore_axis_name="s")
_SC_PARAMS = pltpu.CompilerParams(needs_layout_passes=False,
                                  use_tc_tiling_on_sc=False)


def _deg_body(dst_hbm, out_hbm, dst_v, deg_v):
    c = lax.axis_index("c")
    s = lax.axis_index("s")
    wid = c * NS + s
    pltpu.sync_copy(dst_hbm.at[wid], dst_v)

    def zero(i, carry):
        deg_v[pl.ds(i * L, L)] = jnp.zeros((L,), jnp.float32)
        return carry

    lax.fori_loop(0, N_PAD // L, zero, 0)

    ones = jnp.ones((L,), jnp.float32)

    def body(i, carry):
        j = i // (CHUNK // L)
        k = i % (CHUNK // L)
        idx = dst_v[j, pl.ds(k * L, L)]
        plsc.addupdate_scatter(deg_v, [idx], ones)
        return carry

    lax.fori_loop(0, NCHUNK * (CHUNK // L), body, 0)
    pltpu.sync_copy(deg_v, out_hbm.at[wid])


@functools.partial(
    pl.kernel,
    mesh=_MESH,
    compiler_params=_SC_PARAMS,
    out_type=jax.ShapeDtypeStruct((NW, N_PAD), jnp.float32),
    scratch_types=[
        pltpu.VMEM((NCHUNK, CHUNK), jnp.int32),
        pltpu.VMEM((N_PAD,), jnp.float32),
    ],
)
def _deg_kernel(dst_hbm, out_hbm, dst_v, deg_v):
    _deg_body(dst_hbm, out_hbm, dst_v, deg_v)


NBUF = 4


def _spmm_body(h_hbm, src_hbm, dst_hbm, zeros_hbm, out_hbm,
               src_v, dst_v, bufs, acc, sems, sem_i):
    c = lax.axis_index("c")
    s = lax.axis_index("s")
    wid = c * NS + s

    ci = pltpu.async_copy(src_hbm.at[wid], src_v, sem_i)
    cd = pltpu.async_copy(dst_hbm.at[wid], dst_v, sem_i)
    cz = pltpu.async_copy(zeros_hbm, acc.at[pl.ds(s * RPS, RPS)], sems[0])
    ci.wait()
    cd.wait()
    cz.wait()
    plsc.subcore_barrier()

    def gather(j, u):
        return pltpu.async_copy(h_hbm.at[src_v.at[j]], bufs[u], sems[u])

    for u in range(NBUF - 1):
        gather(u, u)

    def body(t, carry):
        for u in range(NBUF):
            j = NBUF * t + u
            nxt = j + NBUF - 1

            un = (u + NBUF - 1) % NBUF

            @pl.when(nxt < NCHUNK)
            def _():
                gather(nxt, un)

            pltpu.make_async_copy(
                h_hbm.at[src_v.at[j]], bufs[u], sems[u]).wait()
            pltpu.sync_copy(bufs[u], acc.at[dst_v.at[j]], add=True)
        return carry

    lax.fori_loop(0, NCHUNK // NBUF, body, 0)
    plsc.subcore_barrier()
    pltpu.sync_copy(acc.at[pl.ds(s * RPS, RPS)],
                    out_hbm.at[c, pl.ds(s * RPS, RPS)])


@functools.partial(
    pl.kernel,
    mesh=_MESH,
    compiler_params=_SC_PARAMS,
    out_type=jax.ShapeDtypeStruct((NC, N_PAD, W), jnp.float32),
    scratch_types=[
        pltpu.VMEM((NCHUNK, CHUNK), jnp.int32),
        pltpu.VMEM((NCHUNK, CHUNK), jnp.int32),
        [pltpu.VMEM((CHUNK, W), jnp.float32)] * NBUF,
        pltpu.VMEM_SHARED((N_PAD, W), jnp.float32),
        [pltpu.SemaphoreType.DMA] * NBUF,
        pltpu.SemaphoreType.DMA,
    ],
)
def _spmm_kernel(h_hbm, src_hbm, dst_hbm, zeros_hbm, out_hbm,
                 src_v, dst_v, bufs, acc, sems, sem_i):
    _spmm_body(h_hbm, src_hbm, dst_hbm, zeros_hbm, out_hbm,
               src_v, dst_v, bufs, acc, sems, sem_i)


def _tc1_body(x_ref, w1_ref, deg_ref, h_ref, inv_ref):
    dsum = jnp.sum(deg_ref[...], axis=1, keepdims=True)
    inv = lax.rsqrt(jnp.maximum(dsum, 1.0))
    inv_ref[...] = inv
    h_ref[...] = jnp.dot(x_ref[...], w1_ref[...],
                         preferred_element_type=jnp.float32) * inv


def _tc2_body(p0_ref, p1_ref, inv_ref, w_ref, out_ref):
    inv = inv_ref[...]
    hidden = jnp.maximum((p0_ref[...] + p1_ref[...]) * inv, 0.0)
    out_ref[...] = jnp.dot(hidden, w_ref[...],
                           preferred_element_type=jnp.float32) * inv


def _tc3_body(zm0_ref, zm1_ref, zl0_ref, zl1_ref, inv_ref, eps_ref, z_ref):
    inv = inv_ref[...]
    zm = (zm0_ref[...] + zm1_ref[...]) * inv
    zl = (zl0_ref[...] + zl1_ref[...]) * inv
    z_ref[...] = zm + jnp.exp(0.5 * zl) * eps_ref[...]


def kernel(x, adj, W1, W_mu, W_sigma, epsilon):
    src = adj[0]
    dst = adj[1]
    pad = E_PAD - E
    src3 = jnp.concatenate([src, jnp.zeros((pad,), jnp.int32)]).reshape(
        NW, NCHUNK, CHUNK)
    dst3 = jnp.concatenate([dst, jnp.full((pad,), N, jnp.int32)]).reshape(
        NW, NCHUNK, CHUNK)

    zeros_h = jnp.zeros((RPS, W), jnp.float32)
    Wcat = jnp.concatenate([W_mu, W_sigma], axis=1)

    deg_parts = _deg_kernel(dst3)
    degT = deg_parts.T[:N]

    h0s, inv = pl.pallas_call(
        _tc1_body,
        out_shape=[
            jax.ShapeDtypeStruct((N, W), jnp.float32),
            jax.ShapeDtypeStruct((N, 1), jnp.float32),
        ],
    )(x, W1, degT)

    p = _spmm_kernel(h0s, src3, dst3, zeros_h)

    h2s = pl.pallas_call(
        _tc2_body,
        out_shape=jax.ShapeDtypeStruct((N, W), jnp.float32),
    )(p[0, :N], p[1, :N], inv, Wcat)

    q = _spmm_kernel(h2s, src3, dst3, zeros_h)

    Z = pl.pallas_call(
        _tc3_body,
        out_shape=jax.ShapeDtypeStruct((N, H2), jnp.float32),
    )(q[0, :N, :H2], q[1, :N, :H2], q[0, :N, H2:H1], q[1, :N, H2:H1],
      inv, epsilon)
    return Z


# speedup vs baseline: 1.8652x; 1.2186x over previous
import functools

import jax
import jax.numpy as jnp
from jax import lax
from jax.experimental import pallas as pl
from jax.experimental.pallas import tpu as pltpu
from jax.experimental.pallas import tpu_sc as plsc

N = 10000
E = 320000
D_IN = 128
H1 = 64
H2 = 32
W = 64

NC = 2
NS = 16
NW = NC * NS
CHUNK = 128
BLK = 8
NBLK = 10
NCHUNK = BLK * NBLK
E_PAD = NW * CHUNK * NCHUNK
N_PAD = 10240
RPS = N_PAD // NS
L = 16

_MESH = plsc.VectorSubcoreMesh(core_axis_name="c", subcore_axis_name="s")
_SC_PARAMS = pltpu.CompilerParams(needs_layout_passes=False,
                                  use_tc_tiling_on_sc=False)


def _deg_body(dst_hbm, out_hbm, dst_v, deg_v):
    c = lax.axis_index("c")
    s = lax.axis_index("s")
    wid = c * NS + s
    pltpu.sync_copy(dst_hbm.at[wid], dst_v)

    def zero(i, carry):
        deg_v[pl.ds(i * L, L)] = jnp.zeros((L,), jnp.float32)
        return carry

    lax.fori_loop(0, N_PAD // L, zero, 0)

    ones = jnp.ones((L,), jnp.float32)

    def body(i, carry):
        j = i // (CHUNK // L)
        k = i % (CHUNK // L)
        idx = dst_v[j, pl.ds(k * L, L)]
        plsc.addupdate_scatter(deg_v, [idx], ones)
        return carry

    lax.fori_loop(0, NCHUNK * (CHUNK // L), body, 0)
    pltpu.sync_copy(deg_v, out_hbm.at[wid])


@functools.partial(
    pl.kernel,
    mesh=_MESH,
    compiler_params=_SC_PARAMS,
    out_type=jax.ShapeDtypeStruct((NW, N_PAD), jnp.float32),
    scratch_types=[
        pltpu.VMEM((NCHUNK, CHUNK), jnp.int32),
        pltpu.VMEM((N_PAD,), jnp.float32),
    ],
)
def _deg_kernel(dst_hbm, out_hbm, dst_v, deg_v):
    _deg_body(dst_hbm, out_hbm, dst_v, deg_v)


NBUF = 4
K0 = 136
K1 = 24
K_MAX = max(K0, K1)
NCH_TOT = NW * NCHUNK


def _spmm_body(h_hbm, src_hbm, dst_hbm, zeros_hbm, out_hbm,
               src_v, dst_v, bufs, acc, sems, sem_i):
    c = lax.axis_index("c")
    s = lax.axis_index("s")
    base = jnp.where(c == 0, s * K0, NS * K0 + s * K1)
    nch = jnp.where(c == 0, K0, K1)
    nb = jnp.where(c == 0, K0 // NBUF, K1 // NBUF)

    ci = pltpu.async_copy(src_hbm.at[pl.ds(base, K_MAX)], src_v, sem_i)
    cd = pltpu.async_copy(dst_hbm.at[pl.ds(base, K_MAX)], dst_v, sem_i)
    cz = pltpu.async_copy(zeros_hbm, acc.at[pl.ds(s * RPS, RPS)], sems[0])
    ci.wait()
    cd.wait()
    cz.wait()
    plsc.subcore_barrier()

    def gather(j, u):
        return pltpu.async_copy(h_hbm.at[src_v.at[j]], bufs[u], sems[u])

    for u in range(NBUF - 1):
        gather(u, u)

    def body(t, carry):
        for u in range(NBUF):
            j = NBUF * t + u
            nxt = j + NBUF - 1

            un = (u + NBUF - 1) % NBUF

            @pl.when(nxt < nch)
            def _():
                gather(nxt, un)

            pltpu.make_async_copy(
                h_hbm.at[src_v.at[j]], bufs[u], sems[u]).wait()
            pltpu.sync_copy(bufs[u], acc.at[dst_v.at[j]], add=True)
        return carry

    lax.fori_loop(0, nb, body, 0)
    plsc.subcore_barrier()
    pltpu.sync_copy(acc.at[pl.ds(s * RPS, RPS)],
                    out_hbm.at[c, pl.ds(s * RPS, RPS)])


@functools.partial(
    pl.kernel,
    mesh=_MESH,
    compiler_params=_SC_PARAMS,
    out_type=jax.ShapeDtypeStruct((NC, N_PAD, W), jnp.float32),
    scratch_types=[
        pltpu.VMEM((K_MAX, CHUNK), jnp.int32),
        pltpu.VMEM((K_MAX, CHUNK), jnp.int32),
        [pltpu.VMEM((CHUNK, W), jnp.float32)] * NBUF,
        pltpu.VMEM_SHARED((N_PAD, W), jnp.float32),
        [pltpu.SemaphoreType.DMA] * NBUF,
        pltpu.SemaphoreType.DMA,
    ],
)
def _spmm_kernel(h_hbm, src_hbm, dst_hbm, zeros_hbm, out_hbm,
                 src_v, dst_v, bufs, acc, sems, sem_i):
    _spmm_body(h_hbm, src_hbm, dst_hbm, zeros_hbm, out_hbm,
               src_v, dst_v, bufs, acc, sems, sem_i)


def _tc1_body(x_ref, w1_ref, deg_ref, h_ref, inv_ref):
    dsum = jnp.sum(deg_ref[...], axis=1, keepdims=True)
    inv = lax.rsqrt(jnp.maximum(dsum, 1.0))
    inv_ref[...] = inv
    h_ref[...] = jnp.dot(x_ref[...], w1_ref[...],
                         preferred_element_type=jnp.float32) * inv


def _tc2_body(p0_ref, p1_ref, inv_ref, w_ref, out_ref):
    inv = inv_ref[...]
    hidden = jnp.maximum((p0_ref[...] + p1_ref[...]) * inv, 0.0)
    out_ref[...] = jnp.dot(hidden, w_ref[...],
                           preferred_element_type=jnp.float32) * inv


def _tc3_body(zm0_ref, zm1_ref, zl0_ref, zl1_ref, inv_ref, eps_ref, z_ref):
    inv = inv_ref[...]
    zm = (zm0_ref[...] + zm1_ref[...]) * inv
    zl = (zl0_ref[...] + zl1_ref[...]) * inv
    z_ref[...] = zm + jnp.exp(0.5 * zl) * eps_ref[...]


def kernel(x, adj, W1, W_mu, W_sigma, epsilon):
    src = adj[0]
    dst = adj[1]
    pad = E_PAD - E + K_MAX * CHUNK
    src_f = jnp.concatenate([src, jnp.zeros((pad,), jnp.int32)]).reshape(
        NCH_TOT + K_MAX, CHUNK)
    dst_f = jnp.concatenate([dst, jnp.full((pad,), N, jnp.int32)]).reshape(
        NCH_TOT + K_MAX, CHUNK)
    dst3 = dst_f[:NCH_TOT].reshape(NW, NCHUNK, CHUNK)

    zeros_h = jnp.zeros((RPS, W), jnp.float32)
    Wcat = jnp.concatenate([W_mu, W_sigma], axis=1)

    deg_parts = _deg_kernel(dst3)
    degT = deg_parts.T[:N]

    h0s, inv = pl.pallas_call(
        _tc1_body,
        out_shape=[
            jax.ShapeDtypeStruct((N, W), jnp.float32),
            jax.ShapeDtypeStruct((N, 1), jnp.float32),
        ],
    )(x, W1, degT)

    p = _spmm_kernel(h0s, src_f, dst_f, zeros_h)

    h2s = pl.pallas_call(
        _tc2_body,
        out_shape=jax.ShapeDtypeStruct((N, W), jnp.float32),
    )(p[0, :N], p[1, :N], inv, Wcat)

    q = _spmm_kernel(h2s, src_f, dst_f, zeros_h)

    Z = pl.pallas_call(
        _tc3_body,
        out_shape=jax.ShapeDtypeStruct((N, H2), jnp.float32),
    )(q[0, :N, :H2], q[1, :N, :H2], q[0, :N, H2:H1], q[1, :N, H2:H1],
      inv, epsilon)
    return Z


# speedup vs baseline: 1.8911x; 1.0139x over previous
import functools

import jax
import jax.numpy as jnp
from jax import lax
from jax.experimental import pallas as pl
from jax.experimental.pallas import tpu as pltpu
from jax.experimental.pallas import tpu_sc as plsc

N = 10000
E = 320000
D_IN = 128
H1 = 64
H2 = 32
W = 64

NC = 2
NS = 16
NW = NC * NS
CHUNK = 128
BLK = 8
NBLK = 10
NCHUNK = BLK * NBLK
E_PAD = NW * CHUNK * NCHUNK
N_PAD = 10240
RPS = N_PAD // NS
L = 16

_MESH = plsc.VectorSubcoreMesh(core_axis_name="c", subcore_axis_name="s")
_SC_PARAMS = pltpu.CompilerParams(needs_layout_passes=False,
                                  use_tc_tiling_on_sc=False)


def _deg_body(dst_hbm, out_hbm, dst_v, deg_v):
    c = lax.axis_index("c")
    s = lax.axis_index("s")
    wid = c * NS + s
    pltpu.sync_copy(dst_hbm.at[wid], dst_v)

    def zero(i, carry):
        deg_v[pl.ds(i * L, L)] = jnp.zeros((L,), jnp.float32)
        return carry

    lax.fori_loop(0, N_PAD // L, zero, 0)

    ones = jnp.ones((L,), jnp.float32)

    def body(i, carry):
        j = i // (CHUNK // L)
        k = i % (CHUNK // L)
        idx = dst_v[j, pl.ds(k * L, L)]
        plsc.addupdate_scatter(deg_v, [idx], ones)
        return carry

    lax.fori_loop(0, NCHUNK * (CHUNK // L), body, 0)
    pltpu.sync_copy(deg_v, out_hbm.at[wid])


@functools.partial(
    pl.kernel,
    mesh=_MESH,
    compiler_params=_SC_PARAMS,
    out_type=jax.ShapeDtypeStruct((NW, N_PAD), jnp.float32),
    scratch_types=[
        pltpu.VMEM((NCHUNK, CHUNK), jnp.int32),
        pltpu.VMEM((N_PAD,), jnp.float32),
    ],
)
def _deg_kernel(dst_hbm, out_hbm, dst_v, deg_v):
    _deg_body(dst_hbm, out_hbm, dst_v, deg_v)


NBUF = 4
K0 = 136
K1 = 24
K_MAX = max(K0, K1)
NCH_TOT = NW * NCHUNK


def _spmm_body(h_hbm, src_hbm, dst_hbm, out_hbm,
               src_v, dst_v, bufs, acc, sems, sem_i):
    c = lax.axis_index("c")
    s = lax.axis_index("s")
    base = jnp.where(c == 0, s * K0, NS * K0 + s * K1)
    nch = jnp.where(c == 0, K0, K1)
    nb = jnp.where(c == 0, K0 // NBUF, K1 // NBUF)

    ci = pltpu.async_copy(src_hbm.at[pl.ds(base, K_MAX)], src_v, sem_i)
    cd = pltpu.async_copy(dst_hbm.at[pl.ds(base, K_MAX)], dst_v, sem_i)

    def zb(i, carry):
        bufs[0][i // (W // L), pl.ds((i % (W // L)) * L, L)] = (
            jnp.zeros((L,), jnp.float32))
        return carry

    lax.fori_loop(0, CHUNK * (W // L), zb, 0)
    for r in range(RPS // CHUNK):
        pltpu.sync_copy(bufs[0], acc.at[pl.ds(s * RPS + r * CHUNK, CHUNK)])
    ci.wait()
    cd.wait()
    plsc.subcore_barrier()

    def gather(j, u):
        return pltpu.async_copy(h_hbm.at[src_v.at[j]], bufs[u], sems[u])

    for u in range(NBUF - 1):
        gather(u, u)

    def body(t, carry):
        for u in range(NBUF):
            j = NBUF * t + u
            nxt = j + NBUF - 1

            un = (u + NBUF - 1) % NBUF

            @pl.when(nxt < nch)
            def _():
                gather(nxt, un)

            pltpu.make_async_copy(
                h_hbm.at[src_v.at[j]], bufs[u], sems[u]).wait()
            pltpu.sync_copy(bufs[u], acc.at[dst_v.at[j]], add=True)
        return carry

    lax.fori_loop(0, nb, body, 0)
    plsc.subcore_barrier()
    co = pltpu.async_copy(acc.at[pl.ds(s * RPS, CHUNK)], bufs[0], sems[0])
    for r in range(RPS // CHUNK):
        co.wait()
        if r + 1 < RPS // CHUNK:
            nco = pltpu.async_copy(
                acc.at[pl.ds(s * RPS + (r + 1) * CHUNK, CHUNK)],
                bufs[(r + 1) % 2], sems[(r + 1) % 2])
        pltpu.sync_copy(bufs[r % 2],
                        out_hbm.at[c, pl.ds(s * RPS + r * CHUNK, CHUNK)])
        if r + 1 < RPS // CHUNK:
            co = nco


@functools.partial(
    pl.kernel,
    mesh=_MESH,
    compiler_params=_SC_PARAMS,
    out_type=jax.ShapeDtypeStruct((NC, N_PAD, W), jnp.float32),
    scratch_types=[
        pltpu.VMEM((K_MAX, CHUNK), jnp.int32),
        pltpu.VMEM((K_MAX, CHUNK), jnp.int32),
        [pltpu.VMEM((CHUNK, W), jnp.float32)] * NBUF,
        pltpu.VMEM_SHARED((N_PAD, W), jnp.float32),
        [pltpu.SemaphoreType.DMA] * NBUF,
        pltpu.SemaphoreType.DMA,
    ],
)
def _spmm_kernel(h_hbm, src_hbm, dst_hbm, out_hbm,
                 src_v, dst_v, bufs, acc, sems, sem_i):
    _spmm_body(h_hbm, src_hbm, dst_hbm, out_hbm,
               src_v, dst_v, bufs, acc, sems, sem_i)


def _tc1_body(x_ref, w1_ref, deg_ref, h_ref, inv_ref):
    dsum = jnp.sum(deg_ref[...], axis=1, keepdims=True)
    inv = lax.rsqrt(jnp.maximum(dsum, 1.0))
    inv_ref[...] = inv
    h_ref[...] = jnp.dot(x_ref[...], w1_ref[...],
                         preferred_element_type=jnp.float32) * inv


def _tc2_body(p0_ref, p1_ref, inv_ref, w_ref, out_ref):
    inv = inv_ref[...]
    hidden = jnp.maximum((p0_ref[...] + p1_ref[...]) * inv, 0.0)
    out_ref[...] = jnp.dot(hidden, w_ref[...],
                           preferred_element_type=jnp.float32) * inv


def _tc3_body(zm0_ref, zm1_ref, zl0_ref, zl1_ref, inv_ref, eps_ref, z_ref):
    inv = inv_ref[...]
    zm = (zm0_ref[...] + zm1_ref[...]) * inv
    zl = (zl0_ref[...] + zl1_ref[...]) * inv
    z_ref[...] = zm + jnp.exp(0.5 * zl) * eps_ref[...]


def kernel(x, adj, W1, W_mu, W_sigma, epsilon):
    src = adj[0]
    dst = adj[1]
    pad = E_PAD - E + K_MAX * CHUNK
    src_f = jnp.concatenate([src, jnp.zeros((pad,), jnp.int32)]).reshape(
        NCH_TOT + K_MAX, CHUNK)
    dst_f = jnp.concatenate([dst, jnp.full((pad,), N, jnp.int32)]).reshape(
        NCH_TOT + K_MAX, CHUNK)
    dst3 = dst_f[:NCH_TOT].reshape(NW, NCHUNK, CHUNK)

    Wcat = jnp.concatenate([W_mu, W_sigma], axis=1)

    deg_parts = _deg_kernel(dst3)
    degT = deg_parts.T[:N]

    h0s, inv = pl.pallas_call(
        _tc1_body,
        out_shape=[
            jax.ShapeDtypeStruct((N, W), jnp.float32),
            jax.ShapeDtypeStruct((N, 1), jnp.float32),
        ],
    )(x, W1, degT)

    p = _spmm_kernel(h0s, src_f, dst_f)

    h2s = pl.pallas_call(
        _tc2_body,
        out_shape=jax.ShapeDtypeStruct((N, W), jnp.float32),
    )(p[0, :N], p[1, :N], inv, Wcat)

    q = _spmm_kernel(h2s, src_f, dst_f)

    Z = pl.pallas_call(
        _tc3_body,
        out_shape=jax.ShapeDtypeStruct((N, H2), jnp.float32),
    )(q[0, :N, :H2], q[1, :N, :H2], q[0, :N, H2:H1], q[1, :N, H2:H1],
      inv, epsilon)
    return Z
